# baseline (speedup 1.0000x reference)
import functools

import jax
import jax.numpy as jnp
from jax import lax
from jax.experimental import pallas as pl
from jax.experimental.pallas import tpu as pltpu
from jax.experimental.pallas import tpu_sc as plsc

B = 4096
V = 100000
NC = 2
NS = 16
NW = NC * NS
RPW = B // NW
L = 16


def _sc_gather_partials(inp_flat, tgt_flat):
    mesh = plsc.VectorSubcoreMesh(core_axis_name="c", subcore_axis_name="s")

    @functools.partial(
        pl.kernel,
        out_type=jax.ShapeDtypeStruct((NW * L,), jnp.float32),
        mesh=mesh,
        scratch_types=[
            pltpu.VMEM((RPW,), jnp.int32),
            pltpu.VMEM((RPW,), jnp.int32),
            pltpu.VMEM((RPW,), jnp.float32),
            pltpu.VMEM((L,), jnp.float32),
            pltpu.SemaphoreType.DMA,
        ],
    )
    def k(inp_hbm, tgt_hbm, out_hbm, tgt_v, idx_v, val_v, stage_v, sem):
        wid = lax.axis_index("s") * NC + lax.axis_index("c")
        base = wid * RPW
        pltpu.sync_copy(tgt_hbm.at[pl.ds(base, RPW)], tgt_v)
        lanes = lax.iota(jnp.int32, L)
        for j in range(RPW // L):
            t = tgt_v[pl.ds(j * L, L)]
            rows = (base + j * L) + lanes
            idx_v[pl.ds(j * L, L)] = rows * V + t
        pltpu.async_copy(inp_hbm.at[idx_v], val_v, sem).wait()
        acc = jnp.zeros((L,), jnp.float32)
        for j in range(RPW // L):
            t = tgt_v[pl.ds(j * L, L)]
            v = val_v[pl.ds(j * L, L)]
            acc = acc + jnp.where(t > 0, v, 0.0)
        stage_v[...] = acc
        pltpu.sync_copy(stage_v, out_hbm.at[pl.ds(wid * L, L)])

    return k(inp_flat, tgt_flat)


def _reduce_body(p_ref, o_ref):
    o_ref[...] = -jnp.sum(p_ref[...]).reshape(1, 1)


def kernel(input, target):
    tgt = target.reshape(-1).astype(jnp.int32)
    inp_flat = input.reshape(-1)
    partials = _sc_gather_partials(inp_flat, tgt)
    out = pl.pallas_call(
        _reduce_body,
        out_shape=jax.ShapeDtypeStruct((1, 1), jnp.float32),
    )(partials.reshape(4, 128))
    return out[0, 0]


# speedup vs baseline: 2.4016x; 2.4016x over previous
import functools

import jax
import jax.numpy as jnp
from jax import lax
from jax.experimental import pallas as pl
from jax.experimental.pallas import tpu as pltpu
from jax.experimental.pallas import tpu_sc as plsc

B = 4096
V = 100000
NC = 2
NS = 16
NW = NC * NS
RPW = B // NW
L = 16
RND = 64
NR = RPW // RND


def _sc_gather_partials(inp, tgt_flat):
    mesh = plsc.VectorSubcoreMesh(core_axis_name="c", subcore_axis_name="s")

    @functools.partial(
        pl.kernel,
        out_type=jax.ShapeDtypeStruct((NW * L,), jnp.float32),
        mesh=mesh,
        scratch_types=[
            pltpu.VMEM((RPW,), jnp.int32),
            pltpu.VMEM((RND, 8, 128), jnp.float32),
            pltpu.VMEM((L,), jnp.float32),
            pltpu.SemaphoreType.DMA,
        ],
    )
    def k(inp_hbm, tgt_hbm, out_hbm, tgt_v, val_v, stage_v, sem):
        wid = lax.axis_index("s") * NC + lax.axis_index("c")
        base = wid * RPW
        pltpu.sync_copy(tgt_hbm.at[pl.ds(base, RPW)], tgt_v)
        lanes = lax.iota(jnp.int32, L)
        acc = jnp.zeros((L,), jnp.float32)
        for rnd in range(NR):
            r0 = rnd * RND
            tts = []
            for g in range(RND // L):
                tch = tgt_v[pl.ds(r0 + g * L, L)]
                tts.append([tch[j] for j in range(L)])
            copies = []
            for g in range(RND // L):
                for j in range(L):
                    r = r0 + g * L + j
                    ch = pl.multiple_of(tts[g][j] & jnp.int32(~127), 128)
                    copies.append(
                        pltpu.make_async_copy(
                            inp_hbm.at[
                                pl.ds(pl.multiple_of(base + (r // 8) * 8, 8), 8),
                                pl.ds(ch, 128),
                            ],
                            val_v.at[g * L + j],
                            sem,
                        )
                    )
            for cp in copies:
                cp.start()
            for cp in copies:
                cp.wait()
            for g in range(RND // L):
                for j in range(L):
                    r = r0 + g * L + j
                    t = tts[g][j]
                    c16 = pl.multiple_of(t & jnp.int32(112), 16)
                    chunk = val_v[g * L + j, r & 7, pl.ds(c16, L)]
                    tl = jnp.where(t > 0, t & jnp.int32(15), jnp.int32(-1))
                    acc = acc + jnp.where(lanes == tl, chunk, 0.0)
        stage_v[...] = acc
        pltpu.sync_copy(stage_v, out_hbm.at[pl.ds(wid * L, L)])

    return k(inp, tgt_flat)


def _reduce_body(p_ref, o_ref):
    o_ref[...] = -jnp.sum(p_ref[...]).reshape(1, 1)


def kernel(input, target):
    tgt = target.reshape(-1).astype(jnp.int32)
    partials = _sc_gather_partials(input, tgt)
    out = pl.pallas_call(
        _reduce_body,
        out_shape=jax.ShapeDtypeStruct((1, 1), jnp.float32),
    )(partials.reshape(4, 128))
    return out[0, 0]


# speedup vs baseline: 108.1229x; 45.0203x over previous
import functools

import jax
import jax.numpy as jnp
from jax import lax
from jax.experimental import pallas as pl
from jax.experimental.pallas import tpu as pltpu
from jax.experimental.pallas import tpu_sc as plsc

B = 4096
V = 100000
NC = 2
NS = 16
NW = NC * NS
RPW = B // NW
L = 16
RND = 64
NR = RPW // RND


def _sc_gather_partials(inp_t, tgt_flat):
    mesh = plsc.VectorSubcoreMesh(core_axis_name="c", subcore_axis_name="s")

    @functools.partial(
        pl.kernel,
        out_type=jax.ShapeDtypeStruct((NW * L,), jnp.float32),
        mesh=mesh,
        scratch_types=[
            pltpu.VMEM((RPW,), jnp.int32),
            pltpu.VMEM((RND, 8, 128), jnp.float32),
            pltpu.VMEM((L,), jnp.float32),
            pltpu.SemaphoreType.DMA,
        ],
    )
    def k(inp_hbm, tgt_hbm, out_hbm, tgt_v, val_v, stage_v, sem):
        wid = lax.axis_index("s") * NC + lax.axis_index("c")
        base = wid * RPW
        pltpu.sync_copy(tgt_hbm.at[pl.ds(base, RPW)], tgt_v)
        lanes = lax.iota(jnp.int32, L)
        acc = jnp.zeros((L,), jnp.float32)
        for rnd in range(NR):
            r0 = rnd * RND
            tts = []
            for g in range(RND // L):
                tch = tgt_v[pl.ds(r0 + g * L, L)]
                tts.append([tch[j] for j in range(L)])
            copies = []
            for g in range(RND // L):
                for j in range(L):
                    t = tts[g][j]
                    tb = pl.multiple_of(t & jnp.int32(~7), 8)
                    copies.append(
                        pltpu.make_async_copy(
                            inp_hbm.at[
                                pl.ds(tb, 8),
                                pl.ds(pl.multiple_of(base, 128), 128),
                            ],
                            val_v.at[g * L + j],
                            sem,
                        )
                    )
            for cp in copies:
                cp.start()
            for cp in copies:
                cp.wait()
            for g in range(RND // L):
                for j in range(L):
                    r = r0 + g * L + j
                    t = tts[g][j]
                    chunk = val_v[g * L + j, t & jnp.int32(7), pl.ds((r // L) * L, L)]
                    tl = jnp.where(t > 0, jnp.int32(r % L), jnp.int32(-1))
                    acc = acc + jnp.where(lanes == tl, chunk, 0.0)
        stage_v[...] = acc
        pltpu.sync_copy(stage_v, out_hbm.at[pl.ds(wid * L, L)])

    return k(inp_t, tgt_flat)


def _reduce_body(p_ref, o_ref):
    o_ref[...] = -jnp.sum(p_ref[...]).reshape(1, 1)


def kernel(input, target):
    tgt = target.reshape(-1).astype(jnp.int32)
    partials = _sc_gather_partials(input.T, tgt)
    out = pl.pallas_call(
        _reduce_body,
        out_shape=jax.ShapeDtypeStruct((1, 1), jnp.float32),
    )(partials.reshape(4, 128))
    return out[0, 0]


# speedup vs baseline: 143.9929x; 1.3318x over previous
import functools

import jax
import jax.numpy as jnp
from jax import lax
from jax.experimental import pallas as pl
from jax.experimental.pallas import tpu as pltpu
from jax.experimental.pallas import tpu_sc as plsc

B = 4096
V = 100000
NC = 2
NS = 16
NW = NC * NS
RPW = B // NW
L = 16
NG = RPW // L


def _sc_gather_partials(inp_t, tgt_flat):
    mesh = plsc.VectorSubcoreMesh(core_axis_name="c", subcore_axis_name="s")

    @functools.partial(
        pl.kernel,
        out_type=jax.ShapeDtypeStruct((NW * L,), jnp.float32),
        mesh=mesh,
        scratch_types=[
            pltpu.VMEM((RPW,), jnp.int32),
            pltpu.VMEM((RPW, RPW), jnp.float32),
            pltpu.VMEM((L,), jnp.float32),
            pltpu.SemaphoreType.DMA,
        ],
    )
    def k(inp_hbm, tgt_hbm, out_hbm, tgt_v, val_v, stage_v, sem):
        wid = lax.axis_index("s") * NC + lax.axis_index("c")
        base = wid * RPW
        pltpu.sync_copy(tgt_hbm.at[pl.ds(base, RPW)], tgt_v)
        pltpu.async_copy(
            inp_hbm.at[tgt_v, pl.ds(pl.multiple_of(base, 128), RPW)],
            val_v,
            sem,
        ).wait()
        lanes = lax.iota(jnp.int32, L)
        acc = jnp.zeros((L,), jnp.float32)
        for g in range(NG):
            tch = tgt_v[pl.ds(g * L, L)]
            grp = jnp.zeros((L,), jnp.float32)
            for j in range(L):
                r = g * L + j
                chunk = val_v[r, pl.ds(g * L, L)]
                grp = grp + jnp.where(lanes == j, chunk, 0.0)
            acc = acc + jnp.where(tch > 0, grp, 0.0)
        stage_v[...] = acc
        pltpu.sync_copy(stage_v, out_hbm.at[pl.ds(wid * L, L)])

    return k(inp_t, tgt_flat)


def _reduce_body(p_ref, o_ref):
    o_ref[...] = -jnp.sum(p_ref[...]).reshape(1, 1)


def kernel(input, target):
    tgt = target.reshape(-1).astype(jnp.int32)
    partials = _sc_gather_partials(input.T, tgt)
    out = pl.pallas_call(
        _reduce_body,
        out_shape=jax.ShapeDtypeStruct((1, 1), jnp.float32),
    )(partials.reshape(4, 128))
    return out[0, 0]


# speedup vs baseline: 145.1675x; 1.0082x over previous
import functools

import jax
import jax.numpy as jnp
from jax import lax
from jax.experimental import pallas as pl
from jax.experimental.pallas import tpu as pltpu
from jax.experimental.pallas import tpu_sc as plsc

B = 4096
V = 100000
NC = 2
NS = 16
NW = NC * NS
RPW = B // NW
L = 16
NG = RPW // L


def _sc_gather_partials(inp_t, tgt_flat):
    mesh = plsc.VectorSubcoreMesh(core_axis_name="c", subcore_axis_name="s")

    @functools.partial(
        pl.kernel,
        out_type=jax.ShapeDtypeStruct((NW * L,), jnp.float32),
        mesh=mesh,
        scratch_types=[
            pltpu.VMEM((RPW,), jnp.int32),
            pltpu.VMEM((RPW, RPW), jnp.float32),
            pltpu.VMEM((L,), jnp.float32),
            pltpu.SemaphoreType.DMA,
        ],
    )
    def k(inp_hbm, tgt_hbm, out_hbm, tgt_v, val_v, stage_v, sem):
        wid = lax.axis_index("s") * NC + lax.axis_index("c")
        base = wid * RPW
        pltpu.sync_copy(tgt_hbm.at[pl.ds(base, RPW)], tgt_v)
        blk = pl.ds(pl.multiple_of(base, 128), RPW)
        half = RPW // 2
        cps = [
            pltpu.make_async_copy(
                inp_hbm.at[tgt_v.at[pl.ds(h * half, half)], blk],
                val_v.at[pl.ds(h * half, half)],
                sem,
            )
            for h in range(2)
        ]
        cps[0].start()
        cps[1].start()
        lanes = lax.iota(jnp.int32, L)

        def grp_body(g, acc):
            tch = tgt_v[pl.ds(g * L, L)]
            grp = jnp.zeros((L,), jnp.float32)
            for j in range(L):
                chunk = val_v[g * L + j, pl.ds(g * L, L)]
                grp = grp + jnp.where(lanes == j, chunk, 0.0)
            return acc + jnp.where(tch > 0, grp, 0.0)

        cps[0].wait()
        acc = lax.fori_loop(0, NG // 2, grp_body, jnp.zeros((L,), jnp.float32))
        cps[1].wait()
        acc = lax.fori_loop(NG // 2, NG, grp_body, acc)
        stage_v[...] = acc
        pltpu.sync_copy(stage_v, out_hbm.at[pl.ds(wid * L, L)])

    return k(inp_t, tgt_flat)


def _reduce_body(p_ref, o_ref):
    o_ref[...] = -jnp.sum(p_ref[...]).reshape(1, 1)


def kernel(input, target):
    tgt = target.reshape(-1).astype(jnp.int32)
    partials = _sc_gather_partials(input.T, tgt)
    out = pl.pallas_call(
        _reduce_body,
        out_shape=jax.ShapeDtypeStruct((1, 1), jnp.float32),
    )(partials.reshape(4, 128))
    return out[0, 0]


# speedup vs baseline: 145.4617x; 1.0020x over previous
import functools

import jax
import jax.numpy as jnp
from jax import lax
from jax.experimental import pallas as pl
from jax.experimental.pallas import tpu as pltpu
from jax.experimental.pallas import tpu_sc as plsc

B = 4096
V = 100000
NC = 2
NS = 16
NW = NC * NS
RPW = B // NW
L = 16
NG = RPW // L


def _sc_gather_partials(inp_t, tgt_flat):
    mesh = plsc.VectorSubcoreMesh(core_axis_name="c", subcore_axis_name="s")

    @functools.partial(
        pl.kernel,
        out_type=jax.ShapeDtypeStruct((NW * L,), jnp.float32),
        mesh=mesh,
        scratch_types=[
            pltpu.VMEM((RPW,), jnp.int32),
            pltpu.VMEM((RPW, RPW), jnp.float32),
            pltpu.VMEM((L,), jnp.float32),
            pltpu.SemaphoreType.DMA,
        ],
    )
    def k(inp_hbm, tgt_hbm, out_hbm, tgt_v, val_v, stage_v, sem):
        wid = lax.axis_index("s") * NC + lax.axis_index("c")
        base = wid * RPW
        pltpu.sync_copy(tgt_hbm.at[pl.ds(base, RPW)], tgt_v)
        blk = pl.ds(pl.multiple_of(base, 128), RPW)
        half = RPW // 2
        cps = [
            pltpu.make_async_copy(
                inp_hbm.at[tgt_v.at[pl.ds(h * half, half)], blk],
                val_v.at[pl.ds(h * half, half)],
                sem,
            )
            for h in range(2)
        ]
        cps[0].start()
        cps[1].start()
        lanes = lax.iota(jnp.int32, L)

        def grp_body(g, acc):
            tch = tgt_v[pl.ds(g * L, L)]
            grp = jnp.zeros((L,), jnp.float32)
            for j in range(L):
                chunk = val_v[g * L + j, pl.ds(g * L, L)]
                grp = grp + jnp.where(lanes == j, chunk, 0.0)
            return acc + jnp.where(tch > 0, grp, 0.0)

        cps[0].wait()
        acc = plsc.parallel_loop(0, NG // 2, carry=jnp.zeros((L,), jnp.float32))(
            grp_body
        )
        cps[1].wait()
        acc = plsc.parallel_loop(NG // 2, NG, carry=acc)(grp_body)
        stage_v[...] = acc
        pltpu.sync_copy(stage_v, out_hbm.at[pl.ds(wid * L, L)])

    return k(inp_t, tgt_flat)


def _reduce_body(p_ref, o_ref):
    o_ref[...] = -jnp.sum(p_ref[...]).reshape(1, 1)


def kernel(input, target):
    tgt = target.reshape(-1).astype(jnp.int32)
    partials = _sc_gather_partials(input.T, tgt)
    out = pl.pallas_call(
        _reduce_body,
        out_shape=jax.ShapeDtypeStruct((1, 1), jnp.float32),
    )(partials.reshape(4, 128))
    return out[0, 0]


# speedup vs baseline: 145.9717x; 1.0035x over previous
import functools

import jax
import jax.numpy as jnp
from jax import lax
from jax.experimental import pallas as pl
from jax.experimental.pallas import tpu as pltpu
from jax.experimental.pallas import tpu_sc as plsc

B = 4096
V = 100000
NC = 2
NS = 16
NW = NC * NS
RPW = B // NW
L = 16
NG = RPW // L


def _sc_gather_partials(inp_t, tgt_flat):
    mesh = plsc.VectorSubcoreMesh(core_axis_name="c", subcore_axis_name="s")

    @functools.partial(
        pl.kernel,
        out_type=jax.ShapeDtypeStruct((NW * L,), jnp.float32),
        mesh=mesh,
        scratch_types=[
            pltpu.VMEM((RPW,), jnp.int32),
            pltpu.VMEM((RPW, RPW), jnp.float32),
            pltpu.VMEM((L,), jnp.float32),
            pltpu.SemaphoreType.DMA,
            pltpu.SemaphoreType.DMA,
            pltpu.SemaphoreType.DMA,
            pltpu.SemaphoreType.DMA,
            pltpu.SemaphoreType.DMA,
            pltpu.SemaphoreType.DMA,
        ],
    )
    def k(inp_hbm, tgt_hbm, out_hbm, tgt_v, val_v, stage_v, st0, st1, g0, g1, g2, g3):
        wid = lax.axis_index("s") * NC + lax.axis_index("c")
        base = wid * RPW
        half = RPW // 2
        quar = RPW // 4
        tcs = [
            pltpu.make_async_copy(
                tgt_hbm.at[pl.ds(base + h * half, half)],
                tgt_v.at[pl.ds(h * half, half)],
                s,
            )
            for h, s in enumerate([st0, st1])
        ]
        tcs[0].start()
        tcs[1].start()
        blk = pl.ds(pl.multiple_of(base, 128), RPW)
        gsem = [g0, g1, g2, g3]
        cps = [
            pltpu.make_async_copy(
                inp_hbm.at[tgt_v.at[pl.ds(q * quar, quar)], blk],
                val_v.at[pl.ds(q * quar, quar)],
                gsem[q],
            )
            for q in range(4)
        ]
        tcs[0].wait()
        cps[0].start()
        cps[1].start()
        tcs[1].wait()
        cps[2].start()
        cps[3].start()
        lanes = lax.iota(jnp.int32, L)

        def grp_body(g, acc):
            tch = tgt_v[pl.ds(g * L, L)]
            grp = jnp.zeros((L,), jnp.float32)
            for j in range(L):
                chunk = val_v[g * L + j, pl.ds(g * L, L)]
                grp = grp + jnp.where(lanes == j, chunk, 0.0)
            return acc + jnp.where(tch > 0, grp, 0.0)

        acc = jnp.zeros((L,), jnp.float32)
        for q in range(4):
            cps[q].wait()
            acc = plsc.parallel_loop(q * 2, q * 2 + 2, carry=acc)(grp_body)
        stage_v[...] = acc
        pltpu.sync_copy(stage_v, out_hbm.at[pl.ds(wid * L, L)])

    return k(inp_t, tgt_flat)


def _reduce_body(p_ref, o_ref):
    o_ref[...] = -jnp.sum(p_ref[...]).reshape(1, 1)


def kernel(input, target):
    tgt = target.reshape(-1).astype(jnp.int32)
    partials = _sc_gather_partials(input.T, tgt)
    out = pl.pallas_call(
        _reduce_body,
        out_shape=jax.ShapeDtypeStruct((1, 1), jnp.float32),
    )(partials.reshape(4, 128))
    return out[0, 0]
